# baseline (speedup 1.0000x reference)
import functools

import jax
import jax.numpy as jnp
from jax import lax
from jax.experimental import pallas as pl
from jax.experimental.pallas import tpu as pltpu
from jax.experimental.pallas import tpu_sc as plsc

_NUM_CORES = 2
_NUM_SUBCORES = 16
_NUM_WORKERS = _NUM_CORES * _NUM_SUBCORES

_B_CHUNK = 8
_VB = 2000


def _repack_block(t_ref, o_ref):
    x = t_ref[...]
    o_ref[...] = jnp.reshape(jnp.transpose(x), (_VB // 4, 128))


@functools.lru_cache(maxsize=None)
def _make_repack(vocab: int, dim: int):
    assert vocab % _VB == 0
    grid = vocab // _VB
    return pl.pallas_call(
        _repack_block,
        grid=(grid,),
        in_specs=[pl.BlockSpec((dim, _VB), lambda j: (0, j))],
        out_specs=pl.BlockSpec((_VB // 4, 128), lambda j: (j, 0)),
        out_shape=jax.ShapeDtypeStruct((vocab * dim // 128, 128), jnp.float32),
    )


@functools.lru_cache(maxsize=None)
def _make_gather(batch: int, n_fields: int, dim: int):
    assert batch % (_B_CHUNK * _NUM_WORKERS) == 0
    b_per_worker = batch // _NUM_WORKERS
    rows_per_worker = b_per_worker * n_fields
    chunk_rows = _B_CHUNK * n_fields
    n_chunks = b_per_worker // _B_CHUNK
    assert n_chunks >= 4 and n_chunks % 2 == 0
    assert chunk_rows % 16 == 0
    mesh = plsc.VectorSubcoreMesh(core_axis_name="c", subcore_axis_name="s")

    @functools.partial(
        pl.kernel,
        mesh=mesh,
        out_type=jax.ShapeDtypeStruct((batch, n_fields, dim), jnp.float32),
        scratch_types=[
            pltpu.VMEM((rows_per_worker,), jnp.int32),
            pltpu.VMEM((chunk_rows,), jnp.int32),
            pltpu.VMEM((chunk_rows,), jnp.int32),
            pltpu.VMEM((chunk_rows, 128), jnp.float32),
            pltpu.VMEM((chunk_rows, 128), jnp.float32),
            pltpu.VMEM((chunk_rows, dim), jnp.float32),
            pltpu.VMEM((chunk_rows, dim), jnp.float32),
            pltpu.SemaphoreType.DMA,
            pltpu.SemaphoreType.DMA,
            pltpu.SemaphoreType.DMA,
            pltpu.SemaphoreType.DMA,
        ],
        compiler_params=pltpu.CompilerParams(use_tc_tiling_on_sc=True, needs_layout_passes=False),
    )
    def gather_kernel(ids_hbm, t4_hbm, out_hbm, ids_v, ia_v, ib_v,
                      ga_buf, gb_buf, ea_buf, eb_buf, ga, gb, oa, ob):
        wid = lax.axis_index("s") * _NUM_CORES + lax.axis_index("c")
        base_row = wid * rows_per_worker
        base_b = wid * b_per_worker

        pltpu.sync_copy(ids_hbm.at[pl.ds(base_row, rows_per_worker)], ids_v)

        def fire_gather(g, buf, idx4, gsem):
            def split(k, carry):
                v = ids_v[pl.ds(g * chunk_rows + k * 16, 16)]
                idx4[pl.ds(k * 16, 16)] = v >> 2
                return carry

            lax.fori_loop(0, chunk_rows // 16, split, 0)
            pltpu.async_copy(t4_hbm.at[idx4], buf, gsem)

        def drain_gather(buf, gsem):
            pltpu.make_async_copy(
                t4_hbm.at[pl.ds(0, chunk_rows)], buf, gsem).wait()

        lane = lax.broadcasted_iota(jnp.int32, (16,), 0)

        def extract(g, gbuf, ebuf):
            def jblock(k, carry):
                jv = k * 16 + lane
                ov = (ids_v[pl.ds(g * chunk_rows + k * 16, 16)] & 3) * 32
                for c in range(dim):
                    vals = plsc.load_gather(gbuf, [jv, ov + c])
                    plsc.store_scatter(ebuf, [jv, lane * 0 + c], vals)
                return carry

            lax.fori_loop(0, chunk_rows // 16, jblock, 0)

        def fire_out(g, ebuf, osem):
            for i in range(_B_CHUNK):
                pltpu.async_copy(
                    ebuf.at[pl.ds(i * n_fields, n_fields)],
                    out_hbm.at[base_b + g * _B_CHUNK + i],
                    osem)

        def drain_out(ebuf, osem):
            for i in range(_B_CHUNK):
                pltpu.make_async_copy(
                    ebuf.at[pl.ds(0, n_fields)], out_hbm.at[0], osem).wait()

        last = n_chunks - 1

        fire_gather(0, ga_buf, ia_v, ga)
        fire_gather(1, gb_buf, ib_v, gb)
        drain_gather(ga_buf, ga)
        extract(0, ga_buf, ea_buf)
        fire_out(0, ea_buf, oa)

        def pair(k, carry):
            g1 = 2 * k + 1
            fire_gather(g1 + 1, ga_buf, ia_v, ga)
            drain_gather(gb_buf, gb)
            drain_out(ea_buf, oa)
            extract(g1, gb_buf, eb_buf)
            fire_out(g1, eb_buf, ob)
            g2 = 2 * k + 2
            fire_gather(g2 + 1, gb_buf, ib_v, gb)
            drain_gather(ga_buf, ga)
            drain_out(eb_buf, ob)
            extract(g2, ga_buf, ea_buf)
            fire_out(g2, ea_buf, oa)
            return carry

        lax.fori_loop(0, (n_chunks - 2) // 2, pair, 0)

        drain_gather(gb_buf, gb)
        drain_out(ea_buf, oa)
        extract(last, gb_buf, eb_buf)
        fire_out(last, eb_buf, ob)
        drain_out(eb_buf, ob)

    return gather_kernel


def kernel(ids, table):
    batch, n_fields = ids.shape
    vocab, dim = table.shape
    ids_flat = ids.reshape(batch * n_fields).astype(jnp.int32)
    t4 = table.reshape(vocab * dim // 128, 128)
    return _make_gather(batch, n_fields, dim)(ids_flat, t4)


# speedup vs baseline: 1.5874x; 1.5874x over previous
import functools

import jax
import jax.numpy as jnp
from jax import lax
from jax.experimental import pallas as pl
from jax.experimental.pallas import tpu as pltpu
from jax.experimental.pallas import tpu_sc as plsc

EMBEDDING_DIM = 32
ROWS_G = 1024

_NUM_CORES = 2
_NUM_SUBCORES = 16
_NUM_WORKERS = _NUM_CORES * _NUM_SUBCORES


@functools.lru_cache(maxsize=None)
def _make_gather(total_rows: int, dim: int):
    assert total_rows % (ROWS_G * _NUM_WORKERS) == 0
    rows_per_worker = total_rows // _NUM_WORKERS
    groups_per_worker = rows_per_worker // ROWS_G
    assert groups_per_worker >= 3 and groups_per_worker % 2 == 1
    mesh = plsc.VectorSubcoreMesh(core_axis_name="c", subcore_axis_name="s")

    @functools.partial(
        pl.kernel,
        mesh=mesh,
        out_type=jax.ShapeDtypeStruct((total_rows, dim), jnp.float32),
        scratch_types=[
            pltpu.VMEM((rows_per_worker,), jnp.int32),
            pltpu.VMEM((ROWS_G, dim), jnp.float32),
            pltpu.VMEM((ROWS_G, dim), jnp.float32),
            pltpu.SemaphoreType.DMA,
            pltpu.SemaphoreType.DMA,
            pltpu.SemaphoreType.DMA,
            pltpu.SemaphoreType.DMA,
        ],
        compiler_params=pltpu.CompilerParams(use_tc_tiling_on_sc=False),
    )
    def gather_kernel(ids_hbm, table_hbm, out_hbm, idx_v, buf_a, buf_b,
                      ga, gb, oa, ob):
        wid = lax.axis_index("s") * _NUM_CORES + lax.axis_index("c")
        base_row = wid * rows_per_worker
        pltpu.sync_copy(ids_hbm.at[pl.ds(base_row, rows_per_worker)], idx_v)

        def fire_group(g, buf, gsem):
            pltpu.async_copy(
                table_hbm.at[idx_v.at[pl.ds(g * ROWS_G, ROWS_G)]], buf, gsem)

        def drain_gathers(buf, gsem):
            pltpu.make_async_copy(
                table_hbm.at[pl.ds(0, ROWS_G)], buf, gsem).wait()

        def fire_out(g, buf, osem):
            pltpu.async_copy(
                buf, out_hbm.at[pl.ds(base_row + g * ROWS_G, ROWS_G)], osem)

        def drain_out(buf, osem):
            pltpu.make_async_copy(
                buf, out_hbm.at[pl.ds(0, ROWS_G)], osem).wait()

        last = groups_per_worker - 1

        fire_group(0, buf_a, ga)
        fire_group(1, buf_b, gb)
        drain_gathers(buf_a, ga)
        fire_out(0, buf_a, oa)

        def pair(k, carry):
            g1 = 2 * k + 1
            drain_out(buf_a, oa)
            fire_group(g1 + 1, buf_a, ga)
            drain_gathers(buf_b, gb)
            fire_out(g1, buf_b, ob)
            g2 = 2 * k + 2
            drain_out(buf_b, ob)
            fire_group(g2 + 1, buf_b, gb)
            drain_gathers(buf_a, ga)
            fire_out(g2, buf_a, oa)
            return carry

        lax.fori_loop(0, (groups_per_worker - 3) // 2, pair, 0)

        drain_out(buf_a, oa)
        fire_group(last, buf_a, ga)
        drain_gathers(buf_b, gb)
        fire_out(last - 1, buf_b, ob)

        drain_out(buf_b, ob)
        drain_gathers(buf_a, ga)
        fire_out(last, buf_a, oa)
        drain_out(buf_a, oa)

    return gather_kernel


def kernel(ids, table):
    batch, n_fields = ids.shape
    total = batch * n_fields
    ids_flat = ids.reshape(total).astype(jnp.int32)
    out = _make_gather(total, table.shape[1])(ids_flat, table)
    return out.reshape(batch, n_fields, table.shape[1])
